# baseline (speedup 1.0000x reference)
import functools

import numpy as np
import jax
import jax.numpy as jnp
from jax import lax
from jax.experimental import pallas as pl
from jax.experimental.pallas import tpu as pltpu
from jax.experimental.pallas import tpu_sc as plsc

NV = 6890
NPAD = 6912
RR, AA = 3, 6
RA = RR * AA
RAP = 24
MPAD = NPAD * RA
M24 = NPAD * RAP
NW = 32
ROWS_W = MPAD // NW
NV_W = NPAD // NW
CV = 4
CHUNK = CV * RA
CROWS = CV * RAP
NCH = NV_W // CV
RING = 3
OUTER = NCH // RING
INV_S = float(1.0 / np.sqrt(1.0 + 1e-3))

SIG_D = 544
DOWN_D = 64
KP = 128
NROT = 3
BCL = 128

_SC_PARAMS = pltpu.CompilerParams(use_tc_tiling_on_sc=False,
                                  needs_layout_passes=False)


def _make_format():
    mesh = plsc.VectorSubcoreMesh(core_axis_name="c", subcore_axis_name="s")

    def body(bc_hbm, i0h, i1h, i2h, w0h, w1h, w2h,
             buf, si0, si1, si2, sw0, sw1, sw2):
        wid = lax.axis_index("s") * 2 + lax.axis_index("c")
        pltpu.sync_copy(bc_hbm.at[pl.ds(wid * NV_W, NV_W)], buf)
        iota = lax.iota(jnp.int32, 16)
        si = (si0, si1, si2)
        sw = (sw0, sw1, sw2)
        for j in range(3):

            def grp(q, carry, j=j):
                mb = q * 16
                mv = mb + iota
                rl = mv // RA
                col = (mv % RA) * 6 + (2 * j)
                iv = plsc.load_gather(buf, [rl, col])
                wv = plsc.load_gather(buf, [rl, col + 1])
                si[j][pl.ds(mb, 16)] = iv.astype(jnp.int32)
                sw[j][pl.ds(mb, 16)] = wv
                return carry

            lax.fori_loop(0, ROWS_W // 16, grp, 0)
        base = wid * ROWS_W
        for h, v in ((i0h, si0), (i1h, si1), (i2h, si2),
                     (w0h, sw0), (w1h, sw1), (w2h, sw2)):
            pltpu.sync_copy(v, h.at[pl.ds(base, ROWS_W)])

    return pl.kernel(
        body, mesh=mesh,
        out_type=[jax.ShapeDtypeStruct((MPAD,), jnp.int32)] * 3
        + [jax.ShapeDtypeStruct((MPAD,), jnp.float32)] * 3,
        scratch_types=(
            [pltpu.VMEM((NV_W, BCL), jnp.float32)]
            + [pltpu.VMEM((ROWS_W,), jnp.int32) for _ in range(3)]
            + [pltpu.VMEM((ROWS_W,), jnp.float32) for _ in range(3)]
        ),
        compiler_params=_SC_PARAMS,
    )


_format_bc = _make_format()

_PERM = [(r // RA) * RAP + (r % RA) for r in range(CHUNK)]
_RBASES = [0, 16, 32, 48, 56]
_GDN = lax.GatherDimensionNumbers(offset_dims=(), collapsed_slice_dims=(0,),
                                  start_index_map=(0,))


def _lane_splat(v, i):
    idx = jnp.full((16, 1), i, jnp.int32)
    return lax.gather(v, idx, _GDN, (1,),
                      mode=lax.GatherScatterMode.PROMISE_IN_BOUNDS)


def _bf16_halves(x32):
    w = plsc.bitcast(x32, jnp.int32)
    lo = plsc.bitcast(lax.shift_left(w, 16), jnp.float32)
    hi = plsc.bitcast(lax.bitwise_and(w, np.int32(-65536)), jnp.float32)
    return lo, hi


def _make_gather(D):
    mesh = plsc.VectorSubcoreMesh(core_axis_name="c", subcore_axis_name="s")

    def body(x_hbm, i0h, i1h, i2h, w0h, w1h, w2h, out_hbm, *refs):
        (i0, i1, i2, w0, w1, w2,
         ga0, gb0, gc0, ga1, gb1, gc1, ga2, gb2, gc2,
         ov0, ov1, ov2,
         gs0, gs1, gs2, os0, os1, os2) = refs
        G = ((ga0, gb0, gc0), (ga1, gb1, gc1), (ga2, gb2, gc2))
        ov = (ov0, ov1, ov2)
        GS = (gs0, gs1, gs2)
        OS = (os0, os1, os2)
        idx = (i0, i1, i2)
        wts = (w0, w1, w2)
        wid = lax.axis_index("s") * 2 + lax.axis_index("c")
        base = wid * ROWS_W
        for h, v in ((i0h, i0), (i1h, i1), (i2h, i2),
                     (w0h, w0), (w1h, w1), (w2h, w2)):
            pltpu.sync_copy(h.at[pl.ds(base, ROWS_W)], v)
        zeros16 = jnp.zeros((16,), jnp.float32)
        for s in range(RING):
            for v in range(CV):
                for z in range(RAP - RA):
                    for db in range(D // 16):
                        ov[s][v * RAP + RA + z, pl.ds(db * 16, 16)] = zeros16

        def issue_gather(c, s):
            start = c * CHUNK
            for j in range(3):
                pltpu.async_copy(
                    x_hbm.at[idx[j].at[pl.ds(start, CHUNK)]], G[s][j], GS[s])

        def wait_gather(s):
            for j in range(3):
                pltpu.make_async_copy(
                    x_hbm.at[pl.ds(0, CHUNK)], G[s][j], GS[s]).wait()

        def issue_out(c, s):
            g = wid * NCH + c
            pltpu.async_copy(ov[s], out_hbm.at[pl.ds(g * CROWS, CROWS)], OS[s])

        def wait_out(s):
            pltpu.make_async_copy(
                ov[s], out_hbm.at[pl.ds(0, CROWS)], OS[s]).wait()

        def compute(c, s):
            ga, gb, gc = G[s]
            start = c * CHUNK
            for rb in _RBASES:
                wv0 = wts[0][pl.ds(start + rb, 16)]
                wv1 = wts[1][pl.ds(start + rb, 16)]
                wv2 = wts[2][pl.ds(start + rb, 16)]
                for i in range(16):
                    a = _lane_splat(wv0, i)
                    b2 = _lane_splat(wv1, i)
                    c2 = _lane_splat(wv2, i)
                    r = rb + i
                    p = _PERM[r]
                    for db in range(D // 32):
                        sdb = pl.ds(db * 32, 32)
                        a0, a1 = _bf16_halves(ga[r, sdb])
                        b0, b1 = _bf16_halves(gb[r, sdb])
                        c0, c1 = _bf16_halves(gc[r, sdb])
                        ov[s][p, pl.ds(db * 32, 16)] = (
                            a0 * a + b0 * b2 + c0 * c2)
                        ov[s][p, pl.ds(db * 32 + 16, 16)] = (
                            a1 * a + b1 * b2 + c1 * c2)

        issue_gather(0, 0)
        issue_gather(1, 1)

        def outer(k, carry):
            for b in range(RING):
                c = 3 * k + b
                sn = (b + 2) % 3
                if b == 0:
                    @pl.when(k >= 1)
                    def _():
                        wait_out(sn)
                    issue_gather(c + 2, sn)
                else:
                    wait_out(sn)

                    @pl.when(k <= OUTER - 2)
                    def _():
                        issue_gather(c + 2, sn)
                wait_gather(b)
                compute(c, b)
                issue_out(c, b)
            return carry

        lax.fori_loop(0, OUTER, outer, 0)
        wait_out(2)

    return pl.kernel(
        body, mesh=mesh,
        out_type=jax.ShapeDtypeStruct((M24, D), jnp.float32),
        scratch_types=(
            [pltpu.VMEM((ROWS_W,), jnp.int32) for _ in range(3)]
            + [pltpu.VMEM((ROWS_W,), jnp.float32) for _ in range(3)]
            + [pltpu.VMEM((CHUNK, D), jnp.bfloat16) for _ in range(9)]
            + [pltpu.VMEM((CROWS, D), jnp.float32) for _ in range(3)]
            + [pltpu.SemaphoreType.DMA for _ in range(6)]
        ),
        compiler_params=_SC_PARAMS,
    )


_gather128 = _make_gather(KP)


def _down_proj(signal, Wd, b, s, bt):
    BM = 576

    def body(a_ref, w_ref, b_ref, s_ref, bt_ref, o_ref):
        y = jnp.dot(a_ref[...], w_ref[...], preferred_element_type=jnp.float32)
        y = jnp.maximum(y + b_ref[...], 0.0)
        o_ref[...] = (y * s_ref[...] + bt_ref[...]).astype(jnp.bfloat16)

    return pl.pallas_call(
        body,
        grid=(NPAD // BM,),
        in_specs=[
            pl.BlockSpec((BM, SIG_D), lambda i: (i, 0)),
            pl.BlockSpec((SIG_D, KP), lambda i: (0, 0)),
            pl.BlockSpec((1, KP), lambda i: (0, 0)),
            pl.BlockSpec((1, KP), lambda i: (0, 0)),
            pl.BlockSpec((1, KP), lambda i: (0, 0)),
        ],
        out_specs=pl.BlockSpec((BM, KP), lambda i: (i, 0)),
        out_shape=jax.ShapeDtypeStruct((NPAD, KP), jnp.bfloat16),
    )(signal, Wd, b, s, bt)


def _conv_layer(interp24, Tm, btile, s, bt):
    BM = 576
    RB = BM * RAP

    def body(a_ref, t_ref, b_ref, s_ref, bt_ref, o_ref):
        a = a_ref[...].reshape(BM, RAP * KP)
        y = jnp.dot(a.astype(jnp.bfloat16), t_ref[...],
                    preferred_element_type=jnp.float32)
        y = jnp.maximum(y + b_ref[...], 0.0)
        m = jnp.maximum(jnp.maximum(y[:, :KP], y[:, KP:2 * KP]),
                        y[:, 2 * KP:3 * KP])
        o_ref[...] = (m * s_ref[...] + bt_ref[...]).astype(jnp.bfloat16)

    return pl.pallas_call(
        body,
        grid=(NPAD // BM,),
        in_specs=[
            pl.BlockSpec((RB, KP), lambda i: (i, 0)),
            pl.BlockSpec((RAP * KP, NROT * KP), lambda i: (0, 0)),
            pl.BlockSpec((1, NROT * KP), lambda i: (0, 0)),
            pl.BlockSpec((1, KP), lambda i: (0, 0)),
            pl.BlockSpec((1, KP), lambda i: (0, 0)),
        ],
        out_specs=pl.BlockSpec((BM, KP), lambda i: (i, 0)),
        out_shape=jax.ShapeDtypeStruct((NPAD, KP), jnp.bfloat16),
    )(interp24, Tm, btile, s, bt)


def _final_proj(x, Wo, bo):
    BM, BN = 512, 1024

    def body(a_ref, w_ref, b_ref, o_ref):
        o_ref[...] = jnp.dot(a_ref[...], w_ref[...],
                             preferred_element_type=jnp.float32) + b_ref[...]

    return pl.pallas_call(
        body,
        grid=(pl.cdiv(NV, BM), pl.cdiv(NV, BN)),
        in_specs=[
            pl.BlockSpec((BM, KP), lambda i, j: (i, 0)),
            pl.BlockSpec((KP, BN), lambda i, j: (0, j)),
            pl.BlockSpec((1, BN), lambda i, j: (0, j)),
        ],
        out_specs=pl.BlockSpec((BM, BN), lambda i, j: (i, j)),
        out_shape=jax.ShapeDtypeStruct((NV, NV), jnp.float32),
    )(x, Wo, bo)


def _rot_templates(tpl, rd, kp, dp):
    K, Rq, Aq, D = tpl.shape
    tpl = jnp.pad(tpl, ((0, kp - K), (0, 0), (0, 0), (0, dp - D)))
    blk = np.concatenate([np.arange(0, 32, 2), np.arange(1, 32, 2)])
    d_of_c = (np.arange(dp).reshape(-1, 32)[:, blk]).reshape(-1)
    tpl = tpl[:, :, :, d_of_c]
    mats = []
    for rot in range(0, Aq, rd):
        t = jnp.roll(tpl, -rot, axis=2)
        mats.append(t.transpose(1, 2, 3, 0).reshape(Rq * Aq * dp, kp))
    Tm = jnp.concatenate(mats, axis=1)
    return jnp.pad(Tm, ((0, (RAP - RA) * dp), (0, 0))).astype(jnp.bfloat16)


def _pad1(v, n):
    return jnp.pad(v, (0, n - v.shape[0]))


def kernel(signal, bc, W_down, b_down, gamma_down, beta_down,
           templates_0, bias_0, gamma_0, beta_0,
           templates_1, bias_1, gamma_1, beta_1,
           templates_2, bias_2, gamma_2, beta_2,
           W_out, b_out):
    bcp = jnp.pad(bc.reshape(NV, RA * 6), ((0, NPAD - NV), (0, BCL - RA * 6)))
    i0, i1, i2, w0, w1, w2 = _format_bc(bcp)

    x = _down_proj(signal, jnp.pad(W_down, ((0, 0), (0, KP - DOWN_D))),
                   _pad1(b_down, KP).reshape(1, KP),
                   (_pad1(gamma_down, KP) * INV_S).reshape(1, KP),
                   _pad1(beta_down, KP).reshape(1, KP))

    layers = (
        (templates_0, bias_0, gamma_0, beta_0, 2),
        (templates_1, bias_1, gamma_1, beta_1, 2),
        (templates_2, bias_2, gamma_2, beta_2, 2),
    )
    for tpl, b, g, bt, rd in layers:
        Tm = _rot_templates(tpl, rd, KP, KP)
        btile = jnp.tile(_pad1(b, KP), NROT).reshape(1, NROT * KP)
        interp24 = _gather128(x, i0, i1, i2, w0, w1, w2)
        x = _conv_layer(interp24, Tm, btile,
                        (_pad1(g, KP) * INV_S).reshape(1, KP),
                        _pad1(bt, KP).reshape(1, KP))

    return _final_proj(x, W_out.astype(jnp.bfloat16), b_out.reshape(1, NV))


# speedup vs baseline: 1.0018x; 1.0018x over previous
import functools

import numpy as np
import jax
import jax.numpy as jnp
from jax import lax
from jax.experimental import pallas as pl
from jax.experimental.pallas import tpu as pltpu
from jax.experimental.pallas import tpu_sc as plsc

NV = 6890
NPAD = 6912
RR, AA = 3, 6
RA = RR * AA
RAP = 24
MPAD = NPAD * RA
M24 = NPAD * RAP
NW = 32
ROWS_W = MPAD // NW
NV_W = NPAD // NW
CV = 4
CHUNK = CV * RA
CROWS = CV * RAP
NCH = NV_W // CV
RING = 3
OUTER = NCH // RING
INV_S = float(1.0 / np.sqrt(1.0 + 1e-3))

SIG_D = 544
DOWN_D = 64
KP = 128
NROT = 3
BCL = 128

_SC_PARAMS = pltpu.CompilerParams(use_tc_tiling_on_sc=False,
                                  needs_layout_passes=False)


def _make_format():
    mesh = plsc.VectorSubcoreMesh(core_axis_name="c", subcore_axis_name="s")

    def body(bc_hbm, i0h, i1h, i2h, w0h, w1h, w2h,
             buf, si0, si1, si2, sw0, sw1, sw2):
        wid = lax.axis_index("s") * 2 + lax.axis_index("c")
        pltpu.sync_copy(bc_hbm.at[pl.ds(wid * NV_W, NV_W)], buf)
        iota = lax.iota(jnp.int32, 16)
        si = (si0, si1, si2)
        sw = (sw0, sw1, sw2)
        for j in range(3):

            def grp(q, carry, j=j):
                mb = q * 16
                mv = mb + iota
                rl = mv // RA
                col = (mv % RA) * 6 + (2 * j)
                iv = plsc.load_gather(buf, [rl, col])
                wv = plsc.load_gather(buf, [rl, col + 1])
                si[j][pl.ds(mb, 16)] = iv.astype(jnp.int32)
                sw[j][pl.ds(mb, 16)] = wv
                return carry

            lax.fori_loop(0, ROWS_W // 16, grp, 0)
        base = wid * ROWS_W
        for h, v in ((i0h, si0), (i1h, si1), (i2h, si2),
                     (w0h, sw0), (w1h, sw1), (w2h, sw2)):
            pltpu.sync_copy(v, h.at[pl.ds(base, ROWS_W)])

    return pl.kernel(
        body, mesh=mesh,
        out_type=[jax.ShapeDtypeStruct((MPAD,), jnp.int32)] * 3
        + [jax.ShapeDtypeStruct((MPAD,), jnp.float32)] * 3,
        scratch_types=(
            [pltpu.VMEM((NV_W, BCL), jnp.float32)]
            + [pltpu.VMEM((ROWS_W,), jnp.int32) for _ in range(3)]
            + [pltpu.VMEM((ROWS_W,), jnp.float32) for _ in range(3)]
        ),
        compiler_params=_SC_PARAMS,
    )


_format_bc = _make_format()

_PERM = [(r // RA) * RAP + (r % RA) for r in range(CHUNK)]
_RBASES = [0, 16, 32, 48, 56]


def _make_gather(D, h):
    mesh = plsc.VectorSubcoreMesh(core_axis_name="c", subcore_axis_name="s")

    def body(x_hbm, i0h, i1h, i2h, w0h, w1h, w2h, out_hbm, *refs):
        (i0, i1, i2, w0, w1, w2,
         ga0, gb0, gc0, ga1, gb1, gc1, ga2, gb2, gc2,
         ov0, ov1, ov2,
         gs0, gs1, gs2, os0, os1, os2) = refs
        G = ((ga0, gb0, gc0), (ga1, gb1, gc1), (ga2, gb2, gc2))
        ov = (ov0, ov1, ov2)
        GS = (gs0, gs1, gs2)
        OS = (os0, os1, os2)
        idx = (i0, i1, i2)
        wts = (w0, w1, w2)
        wid = lax.axis_index("s") * 2 + lax.axis_index("c")
        base = h * (MPAD // 2) + wid * ROWS_WH
        for hr, v in ((i0h, i0), (i1h, i1), (i2h, i2),
                     (w0h, w0), (w1h, w1), (w2h, w2)):
            pltpu.sync_copy(hr.at[pl.ds(base, ROWS_WH)], v)
        zeros16 = jnp.zeros((16,), jnp.float32)
        for s in range(RING):
            for v in range(CV):
                for z in range(RAP - RA):
                    for db in range(D // 16):
                        ov[s][v * RAP + RA + z, pl.ds(db * 16, 16)] = zeros16

        def issue_gather(c, s):
            start = c * CHUNK
            for j in range(3):
                pltpu.async_copy(
                    x_hbm.at[idx[j].at[pl.ds(start, CHUNK)]], G[s][j], GS[s])

        def wait_gather(s):
            for j in range(3):
                pltpu.make_async_copy(
                    x_hbm.at[pl.ds(0, CHUNK)], G[s][j], GS[s]).wait()

        def issue_out(c, s):
            g = wid * NCHH + c
            pltpu.async_copy(ov[s], out_hbm.at[pl.ds(g * CROWS, CROWS)], OS[s])

        def wait_out(s):
            pltpu.make_async_copy(
                ov[s], out_hbm.at[pl.ds(0, CROWS)], OS[s]).wait()

        def compute(c, s):
            ga, gb, gc = G[s]
            start = c * CHUNK
            for rb in _RBASES:
                wv0 = wts[0][pl.ds(start + rb, 16)]
                wv1 = wts[1][pl.ds(start + rb, 16)]
                wv2 = wts[2][pl.ds(start + rb, 16)]
                for i in range(16):
                    a, b2, c2 = wv0[i], wv1[i], wv2[i]
                    r = rb + i
                    p = _PERM[r]
                    for db in range(D // 32):
                        sdb = pl.ds(db * 32, 32)
                        a0, a1 = plsc.unpack(
                            ga[r, sdb], format=plsc.PackFormat.INTERLEAVED)
                        b0, b1 = plsc.unpack(
                            gb[r, sdb], format=plsc.PackFormat.INTERLEAVED)
                        c0, c1 = plsc.unpack(
                            gc[r, sdb], format=plsc.PackFormat.INTERLEAVED)
                        ov[s][p, pl.ds(db * 32, 16)] = (
                            a0 * a + b0 * b2 + c0 * c2)
                        ov[s][p, pl.ds(db * 32 + 16, 16)] = (
                            a1 * a + b1 * b2 + c1 * c2)

        issue_gather(0, 0)
        issue_gather(1, 1)

        def outer(k, carry):
            for b in range(RING):
                c = 3 * k + b
                sn = (b + 2) % 3
                if b == 0:
                    @pl.when(k >= 1)
                    def _():
                        wait_out(sn)
                    issue_gather(c + 2, sn)
                else:
                    wait_out(sn)

                    @pl.when(k <= OUTERH - 2)
                    def _():
                        issue_gather(c + 2, sn)
                wait_gather(b)
                compute(c, b)
                issue_out(c, b)
            return carry

        lax.fori_loop(0, OUTERH, outer, 0)
        wait_out(2)

    return pl.kernel(
        body, mesh=mesh,
        out_type=jax.ShapeDtypeStruct((M24 // 2, D), jnp.float32),
        scratch_types=(
            [pltpu.VMEM((ROWS_WH,), jnp.int32) for _ in range(3)]
            + [pltpu.VMEM((ROWS_WH,), jnp.float32) for _ in range(3)]
            + [pltpu.VMEM((CHUNK, D), jnp.bfloat16) for _ in range(9)]
            + [pltpu.VMEM((CROWS, D), jnp.float32) for _ in range(3)]
            + [pltpu.SemaphoreType.DMA for _ in range(6)]
        ),
        compiler_params=_SC_PARAMS,
    )


ROWS_WH = ROWS_W // 2
NV_WH = NV_W // 2
NCHH = NV_WH // CV
OUTERH = NCHH // RING

_gatherA = _make_gather(KP, 0)
_gatherB = _make_gather(KP, 1)


def _down_proj(signal, Wd, b, s, bt):
    BM = 576

    def body(a_ref, w_ref, b_ref, s_ref, bt_ref, o_ref):
        y = jnp.dot(a_ref[...], w_ref[...], preferred_element_type=jnp.float32)
        y = jnp.maximum(y + b_ref[...], 0.0)
        o_ref[...] = (y * s_ref[...] + bt_ref[...]).astype(jnp.bfloat16)

    return pl.pallas_call(
        body,
        grid=(NPAD // BM,),
        in_specs=[
            pl.BlockSpec((BM, SIG_D), lambda i: (i, 0)),
            pl.BlockSpec((SIG_D, KP), lambda i: (0, 0)),
            pl.BlockSpec((1, KP), lambda i: (0, 0)),
            pl.BlockSpec((1, KP), lambda i: (0, 0)),
            pl.BlockSpec((1, KP), lambda i: (0, 0)),
        ],
        out_specs=pl.BlockSpec((BM, KP), lambda i: (i, 0)),
        out_shape=jax.ShapeDtypeStruct((NPAD, KP), jnp.bfloat16),
    )(signal, Wd, b, s, bt)


def _conv_layer(interp24, Tm, btile, s, bt):
    BM = 576
    RB = BM * RAP
    NB = interp24.shape[0] // RB

    def body(a_ref, t_ref, b_ref, s_ref, bt_ref, o_ref):
        a = a_ref[...].reshape(BM, RAP * KP)
        y = jnp.dot(a.astype(jnp.bfloat16), t_ref[...],
                    preferred_element_type=jnp.float32)
        y = jnp.maximum(y + b_ref[...], 0.0)
        m = jnp.maximum(jnp.maximum(y[:, :KP], y[:, KP:2 * KP]),
                        y[:, 2 * KP:3 * KP])
        o_ref[...] = (m * s_ref[...] + bt_ref[...]).astype(jnp.bfloat16)

    return pl.pallas_call(
        body,
        grid=(NB,),
        in_specs=[
            pl.BlockSpec((RB, KP), lambda i: (i, 0)),
            pl.BlockSpec((RAP * KP, NROT * KP), lambda i: (0, 0)),
            pl.BlockSpec((1, NROT * KP), lambda i: (0, 0)),
            pl.BlockSpec((1, KP), lambda i: (0, 0)),
            pl.BlockSpec((1, KP), lambda i: (0, 0)),
        ],
        out_specs=pl.BlockSpec((BM, KP), lambda i: (i, 0)),
        out_shape=jax.ShapeDtypeStruct((NB * BM, KP), jnp.bfloat16),
    )(interp24, Tm, btile, s, bt)


def _final_proj(x, Wo, bo):
    BM, BN = 512, 1024

    def body(a_ref, w_ref, b_ref, o_ref):
        o_ref[...] = jnp.dot(a_ref[...], w_ref[...],
                             preferred_element_type=jnp.float32) + b_ref[...]

    return pl.pallas_call(
        body,
        grid=(pl.cdiv(NV, BM), pl.cdiv(NV, BN)),
        in_specs=[
            pl.BlockSpec((BM, KP), lambda i, j: (i, 0)),
            pl.BlockSpec((KP, BN), lambda i, j: (0, j)),
            pl.BlockSpec((1, BN), lambda i, j: (0, j)),
        ],
        out_specs=pl.BlockSpec((BM, BN), lambda i, j: (i, j)),
        out_shape=jax.ShapeDtypeStruct((NV, NV), jnp.float32),
    )(x, Wo, bo)


def _rot_templates(tpl, rd, kp, dp):
    K, Rq, Aq, D = tpl.shape
    tpl = jnp.pad(tpl, ((0, kp - K), (0, 0), (0, 0), (0, dp - D)))
    blk = np.concatenate([np.arange(0, 32, 2), np.arange(1, 32, 2)])
    d_of_c = (np.arange(dp).reshape(-1, 32)[:, blk]).reshape(-1)
    tpl = tpl[:, :, :, d_of_c]
    mats = []
    for rot in range(0, Aq, rd):
        t = jnp.roll(tpl, -rot, axis=2)
        mats.append(t.transpose(1, 2, 3, 0).reshape(Rq * Aq * dp, kp))
    Tm = jnp.concatenate(mats, axis=1)
    return jnp.pad(Tm, ((0, (RAP - RA) * dp), (0, 0))).astype(jnp.bfloat16)


def _pad1(v, n):
    return jnp.pad(v, (0, n - v.shape[0]))


def kernel(signal, bc, W_down, b_down, gamma_down, beta_down,
           templates_0, bias_0, gamma_0, beta_0,
           templates_1, bias_1, gamma_1, beta_1,
           templates_2, bias_2, gamma_2, beta_2,
           W_out, b_out):
    bcp = jnp.pad(bc.reshape(NV, RA * 6), ((0, NPAD - NV), (0, BCL - RA * 6)))
    i0, i1, i2, w0, w1, w2 = _format_bc(bcp)

    x = _down_proj(signal, jnp.pad(W_down, ((0, 0), (0, KP - DOWN_D))),
                   _pad1(b_down, KP).reshape(1, KP),
                   (_pad1(gamma_down, KP) * INV_S).reshape(1, KP),
                   _pad1(beta_down, KP).reshape(1, KP))

    layers = (
        (templates_0, bias_0, gamma_0, beta_0, 2),
        (templates_1, bias_1, gamma_1, beta_1, 2),
        (templates_2, bias_2, gamma_2, beta_2, 2),
    )
    for tpl, b, g, bt, rd in layers:
        Tm = _rot_templates(tpl, rd, KP, KP)
        btile = jnp.tile(_pad1(b, KP), NROT).reshape(1, NROT * KP)
        sc = (_pad1(g, KP) * INV_S).reshape(1, KP)
        bb = _pad1(bt, KP).reshape(1, KP)
        iA = _gatherA(x, i0, i1, i2, w0, w1, w2)
        iB = _gatherB(x, i0, i1, i2, w0, w1, w2)
        xA = _conv_layer(iA, Tm, btile, sc, bb)
        xB = _conv_layer(iB, Tm, btile, sc, bb)
        x = jnp.concatenate([xA, xB], axis=0)

    return _final_proj(x, W_out.astype(jnp.bfloat16), b_out.reshape(1, NV))


# speedup vs baseline: 1.0287x; 1.0268x over previous
import functools

import numpy as np
import jax
import jax.numpy as jnp
from jax import lax
from jax.experimental import pallas as pl
from jax.experimental.pallas import tpu as pltpu
from jax.experimental.pallas import tpu_sc as plsc

NV = 6890
NPAD = 6912
RR, AA = 3, 6
RA = RR * AA
RAP = 24
MPAD = NPAD * RA
M24 = NPAD * RAP
NW = 32
ROWS_W = MPAD // NW
NV_W = NPAD // NW
CV = 4
CHUNK = CV * RA
CROWS = CV * RAP
NCH = NV_W // CV
RING = 3
OUTER = NCH // RING
INV_S = float(1.0 / np.sqrt(1.0 + 1e-3))

SIG_D = 544
DOWN_D = 64
KP = 128
NROT = 3
BCL = 128

_SC_PARAMS = pltpu.CompilerParams(use_tc_tiling_on_sc=False,
                                  needs_layout_passes=False)


def _make_format():
    mesh = plsc.VectorSubcoreMesh(core_axis_name="c", subcore_axis_name="s")

    def body(bc_hbm, i0h, i1h, i2h, w0h, w1h, w2h,
             buf, si0, si1, si2, sw0, sw1, sw2):
        wid = lax.axis_index("s") * 2 + lax.axis_index("c")
        pltpu.sync_copy(bc_hbm.at[pl.ds(wid * NV_W, NV_W)], buf)
        iota = lax.iota(jnp.int32, 16)
        si = (si0, si1, si2)
        sw = (sw0, sw1, sw2)
        for j in range(3):

            def grp(q, carry, j=j):
                mb = q * 16
                mv = mb + iota
                rl = mv // RA
                col = (mv % RA) * 6 + (2 * j)
                iv = plsc.load_gather(buf, [rl, col])
                wv = plsc.load_gather(buf, [rl, col + 1])
                si[j][pl.ds(mb, 16)] = iv.astype(jnp.int32)
                sw[j][pl.ds(mb, 16)] = wv
                return carry

            lax.fori_loop(0, ROWS_W // 16, grp, 0)
        base = wid * ROWS_W
        for h, v in ((i0h, si0), (i1h, si1), (i2h, si2),
                     (w0h, sw0), (w1h, sw1), (w2h, sw2)):
            pltpu.sync_copy(v, h.at[pl.ds(base, ROWS_W)])

    return pl.kernel(
        body, mesh=mesh,
        out_type=[jax.ShapeDtypeStruct((MPAD,), jnp.int32)] * 3
        + [jax.ShapeDtypeStruct((MPAD,), jnp.float32)] * 3,
        scratch_types=(
            [pltpu.VMEM((NV_W, BCL), jnp.float32)]
            + [pltpu.VMEM((ROWS_W,), jnp.int32) for _ in range(3)]
            + [pltpu.VMEM((ROWS_W,), jnp.float32) for _ in range(3)]
        ),
        compiler_params=_SC_PARAMS,
    )


_format_bc = _make_format()

_PERM = [(r // RA) * RAP + (r % RA) for r in range(CHUNK)]
_RBASES = [0, 16, 32, 48, 56]


def _make_gather(D):
    mesh = plsc.VectorSubcoreMesh(core_axis_name="c", subcore_axis_name="s")

    def body(x_hbm, i0h, i1h, i2h, w0h, w1h, w2h, out_hbm, *refs):
        (i0, i1, i2, w0, w1, w2,
         ga0, gb0, gc0, ga1, gb1, gc1, ga2, gb2, gc2,
         ov0, ov1, ov2,
         gs0, gs1, gs2, os0, os1, os2) = refs
        G = ((ga0, gb0, gc0), (ga1, gb1, gc1), (ga2, gb2, gc2))
        ov = (ov0, ov1, ov2)
        GS = (gs0, gs1, gs2)
        OS = (os0, os1, os2)
        idx = (i0, i1, i2)
        wts = (w0, w1, w2)
        wid = lax.axis_index("s") * 2 + lax.axis_index("c")
        base = wid * ROWS_W
        for h, v in ((i0h, i0), (i1h, i1), (i2h, i2),
                     (w0h, w0), (w1h, w1), (w2h, w2)):
            pltpu.sync_copy(h.at[pl.ds(base, ROWS_W)], v)
        zeros16 = jnp.zeros((16,), jnp.float32)
        for s in range(RING):
            for v in range(CV):
                for z in range(RAP - RA):
                    for db in range(D // 16):
                        ov[s][v * RAP + RA + z, pl.ds(db * 16, 16)] = zeros16

        def issue_gather(c, s):
            start = c * CHUNK
            for j in range(3):
                pltpu.async_copy(
                    x_hbm.at[idx[j].at[pl.ds(start, CHUNK)]], G[s][j], GS[s])

        def wait_gather(s):
            for j in range(3):
                pltpu.make_async_copy(
                    x_hbm.at[pl.ds(0, CHUNK)], G[s][j], GS[s]).wait()

        def issue_out(c, s):
            g = wid * NCH + c
            pltpu.async_copy(ov[s], out_hbm.at[pl.ds(g * CROWS, CROWS)], OS[s])

        def wait_out(s):
            pltpu.make_async_copy(
                ov[s], out_hbm.at[pl.ds(0, CROWS)], OS[s]).wait()

        def compute(c, s):
            ga, gb, gc = G[s]
            start = c * CHUNK
            for rb in _RBASES:
                wv0 = wts[0][pl.ds(start + rb, 16)]
                wv1 = wts[1][pl.ds(start + rb, 16)]
                wv2 = wts[2][pl.ds(start + rb, 16)]
                for i in range(16):
                    a, b2, c2 = wv0[i], wv1[i], wv2[i]
                    r = rb + i
                    p = _PERM[r]
                    for db in range(D // 32):
                        sdb = pl.ds(db * 32, 32)
                        a0, a1 = plsc.unpack(
                            ga[r, sdb], format=plsc.PackFormat.INTERLEAVED)
                        b0, b1 = plsc.unpack(
                            gb[r, sdb], format=plsc.PackFormat.INTERLEAVED)
                        c0, c1 = plsc.unpack(
                            gc[r, sdb], format=plsc.PackFormat.INTERLEAVED)
                        ov[s][p, pl.ds(db * 32, 16)] = (
                            a0 * a + b0 * b2 + c0 * c2)
                        ov[s][p, pl.ds(db * 32 + 16, 16)] = (
                            a1 * a + b1 * b2 + c1 * c2)

        issue_gather(0, 0)
        issue_gather(1, 1)

        def outer(k, carry):
            for b in range(RING):
                c = 3 * k + b
                sn = (b + 2) % 3
                if b == 0:
                    @pl.when(k >= 1)
                    def _():
                        wait_out(sn)
                    issue_gather(c + 2, sn)
                else:
                    wait_out(sn)

                    @pl.when(k <= OUTER - 2)
                    def _():
                        issue_gather(c + 2, sn)
                wait_gather(b)
                compute(c, b)
                issue_out(c, b)
            return carry

        lax.fori_loop(0, OUTER, outer, 0)
        wait_out(2)

    return pl.kernel(
        body, mesh=mesh,
        out_type=jax.ShapeDtypeStruct((M24, D), jnp.float32),
        scratch_types=(
            [pltpu.VMEM((ROWS_W,), jnp.int32) for _ in range(3)]
            + [pltpu.VMEM((ROWS_W,), jnp.float32) for _ in range(3)]
            + [pltpu.VMEM((CHUNK, D), jnp.bfloat16) for _ in range(9)]
            + [pltpu.VMEM((CROWS, D), jnp.float32) for _ in range(3)]
            + [pltpu.SemaphoreType.DMA for _ in range(6)]
        ),
        compiler_params=_SC_PARAMS,
    )


_gather128 = _make_gather(KP)


def _down_proj(signal, Wd, b, s, bt):
    BM = 576

    def body(a_ref, w_ref, b_ref, s_ref, bt_ref, o_ref):
        y = jnp.dot(a_ref[...], w_ref[...], preferred_element_type=jnp.float32)
        y = jnp.maximum(y + b_ref[...], 0.0)
        o_ref[...] = (y * s_ref[...] + bt_ref[...]).astype(jnp.bfloat16)

    return pl.pallas_call(
        body,
        grid=(NPAD // BM,),
        in_specs=[
            pl.BlockSpec((BM, SIG_D), lambda i: (i, 0)),
            pl.BlockSpec((SIG_D, KP), lambda i: (0, 0)),
            pl.BlockSpec((1, KP), lambda i: (0, 0)),
            pl.BlockSpec((1, KP), lambda i: (0, 0)),
            pl.BlockSpec((1, KP), lambda i: (0, 0)),
        ],
        out_specs=pl.BlockSpec((BM, KP), lambda i: (i, 0)),
        out_shape=jax.ShapeDtypeStruct((NPAD, KP), jnp.bfloat16),
    )(signal, Wd, b, s, bt)


def _conv_layer(interp24, Tm, btile, s, bt):
    BM = 576
    RB = BM * RAP

    def body(a_ref, t_ref, b_ref, s_ref, bt_ref, o_ref):
        a = a_ref[...].reshape(BM, RAP * KP)
        y = jnp.dot(a.astype(jnp.bfloat16), t_ref[...],
                    preferred_element_type=jnp.float32)
        y = jnp.maximum(y + b_ref[...], 0.0)
        m = jnp.maximum(jnp.maximum(y[:, :KP], y[:, KP:2 * KP]),
                        y[:, 2 * KP:3 * KP])
        o_ref[...] = (m * s_ref[...] + bt_ref[...]).astype(jnp.bfloat16)

    return pl.pallas_call(
        body,
        grid=(NPAD // BM,),
        in_specs=[
            pl.BlockSpec((RB, KP), lambda i: (i, 0)),
            pl.BlockSpec((RAP * KP, NROT * KP), lambda i: (0, 0)),
            pl.BlockSpec((1, NROT * KP), lambda i: (0, 0)),
            pl.BlockSpec((1, KP), lambda i: (0, 0)),
            pl.BlockSpec((1, KP), lambda i: (0, 0)),
        ],
        out_specs=pl.BlockSpec((BM, KP), lambda i: (i, 0)),
        out_shape=jax.ShapeDtypeStruct((NPAD, KP), jnp.bfloat16),
    )(interp24, Tm, btile, s, bt)


def _final_proj(x, Wo, bo):
    BM, BN = 512, 1024

    def body(a_ref, w_ref, b_ref, o_ref):
        o_ref[...] = jnp.dot(a_ref[...], w_ref[...],
                             preferred_element_type=jnp.float32) + b_ref[...]

    return pl.pallas_call(
        body,
        grid=(pl.cdiv(NV, BM), pl.cdiv(NV, BN)),
        in_specs=[
            pl.BlockSpec((BM, KP), lambda i, j: (i, 0)),
            pl.BlockSpec((KP, BN), lambda i, j: (0, j)),
            pl.BlockSpec((1, BN), lambda i, j: (0, j)),
        ],
        out_specs=pl.BlockSpec((BM, BN), lambda i, j: (i, j)),
        out_shape=jax.ShapeDtypeStruct((NV, NV), jnp.float32),
    )(x, Wo, bo)


def _rot_templates(tpl, rd, kp, dp):
    K, Rq, Aq, D = tpl.shape
    tpl = jnp.pad(tpl, ((0, kp - K), (0, 0), (0, 0), (0, dp - D)))
    blk = np.concatenate([np.arange(0, 32, 2), np.arange(1, 32, 2)])
    d_of_c = (np.arange(dp).reshape(-1, 32)[:, blk]).reshape(-1)
    tpl = tpl[:, :, :, d_of_c]
    mats = []
    for rot in range(0, Aq, rd):
        t = jnp.roll(tpl, -rot, axis=2)
        mats.append(t.transpose(1, 2, 3, 0).reshape(Rq * Aq * dp, kp))
    Tm = jnp.concatenate(mats, axis=1)
    return jnp.pad(Tm, ((0, (RAP - RA) * dp), (0, 0))).astype(jnp.bfloat16)


def _pad1(v, n):
    return jnp.pad(v, (0, n - v.shape[0]))


def kernel(signal, bc, W_down, b_down, gamma_down, beta_down,
           templates_0, bias_0, gamma_0, beta_0,
           templates_1, bias_1, gamma_1, beta_1,
           templates_2, bias_2, gamma_2, beta_2,
           W_out, b_out):
    bcp = jnp.pad(bc.reshape(NV, RA * 6), ((0, NPAD - NV), (0, BCL - RA * 6)))
    i0, i1, i2, w0, w1, w2 = _format_bc(bcp)

    x = _down_proj(signal, jnp.pad(W_down, ((0, 0), (0, KP - DOWN_D))),
                   _pad1(b_down, KP).reshape(1, KP),
                   (_pad1(gamma_down, KP) * INV_S).reshape(1, KP),
                   _pad1(beta_down, KP).reshape(1, KP))

    layers = (
        (templates_0, bias_0, gamma_0, beta_0, 2),
        (templates_1, bias_1, gamma_1, beta_1, 2),
        (templates_2, bias_2, gamma_2, beta_2, 2),
    )
    for tpl, b, g, bt, rd in layers:
        Tm = _rot_templates(tpl, rd, KP, KP)
        btile = jnp.tile(_pad1(b, KP), NROT).reshape(1, NROT * KP)
        interp24 = _gather128(x, i0, i1, i2, w0, w1, w2)
        x = _conv_layer(interp24, Tm, btile,
                        (_pad1(g, KP) * INV_S).reshape(1, KP),
                        _pad1(bt, KP).reshape(1, KP))

    return _final_proj(x, W_out.astype(jnp.bfloat16), b_out.reshape(1, NV))
